# baseline (device time: 32399 ns/iter reference)
import jax
import jax.numpy as jnp
from jax import lax
from jax.experimental import pallas as pl
from jax.experimental.pallas import tpu as pltpu

N_DEV = 4


def kernel(A, B):
    m, _ = A.shape
    _, n = B.shape
    ch = m // N_DEV
    hf = ch // 2

    FROM_L, FROM_R = 0, 1

    def body(a_ref, b_ref, out_ref, acc_ref, rs_half_ref, rs_full_ref,
             rs_half_send, rs_half_recv, rs_full_send, rs_full_recv,
             ag_full_send, ag_full_recv, ag_half_send, ag_half_recv):
        my = lax.axis_index("i")
        left = (my + N_DEV - 1) % N_DEV
        right = (my + 1) % N_DEV
        diag = (my + 2) % N_DEV

        barrier_sem = pltpu.get_barrier_semaphore()
        for nbr in (left, right):
            pl.semaphore_signal(
                barrier_sem, inc=1,
                device_id=(nbr,), device_id_type=pl.DeviceIdType.MESH,
            )
        pl.semaphore_wait(barrier_sem, 2)

        b16 = b_ref[...].astype(jnp.bfloat16)

        def partial_chunk(p):
            return jnp.dot(
                a_ref[pl.ds(p * ch, ch), :].astype(jnp.bfloat16),
                b16,
                preferred_element_type=jnp.float32,
            )

        sends = []

        def remote_copy(src, dst, ssem, rsem, dev):
            rdma = pltpu.make_async_remote_copy(
                src_ref=src, dst_ref=dst, send_sem=ssem, recv_sem=rsem,
                device_id=(dev,), device_id_type=pl.DeviceIdType.MESH,
            )
            rdma.start()
            sends.append(rdma)

        def wait_recv(buf, sem):
            pltpu.make_async_remote_copy(
                src_ref=buf, dst_ref=buf, send_sem=sem, recv_sem=sem,
                device_id=(my,), device_id_type=pl.DeviceIdType.MESH,
            ).wait_recv()

        acc_ref[pl.ds(diag * ch, ch), :] = partial_chunk(diag).astype(
            jnp.bfloat16)
        remote_copy(acc_ref.at[pl.ds(diag * ch, hf), :],
                    rs_half_ref.at[FROM_R],
                    rs_half_send.at[0], rs_half_recv.at[FROM_R], left)
        remote_copy(acc_ref.at[pl.ds(diag * ch + hf, hf), :],
                    rs_half_ref.at[FROM_L],
                    rs_half_send.at[1], rs_half_recv.at[FROM_L], right)

        acc_ref[pl.ds(right * ch, ch), :] = partial_chunk(right).astype(
            jnp.bfloat16)
        remote_copy(acc_ref.at[pl.ds(right * ch, hf), :],
                    rs_full_ref.at[FROM_L, pl.ds(0, hf)],
                    rs_full_send.at[0], rs_full_recv.at[0], right)

        wait_recv(rs_half_ref.at[FROM_L], rs_half_recv.at[FROM_L])
        hi = right * ch + hf
        acc_ref[pl.ds(hi, hf), :] = (
            acc_ref[pl.ds(hi, hf), :].astype(jnp.float32)
            + rs_half_ref[FROM_L].astype(jnp.float32)
        ).astype(jnp.bfloat16)
        remote_copy(acc_ref.at[pl.ds(hi, hf), :],
                    rs_full_ref.at[FROM_L, pl.ds(hf, hf)],
                    rs_full_send.at[1], rs_full_recv.at[1], right)

        acc_ref[pl.ds(left * ch, ch), :] = partial_chunk(left).astype(
            jnp.bfloat16)
        remote_copy(acc_ref.at[pl.ds(left * ch + hf, hf), :],
                    rs_full_ref.at[FROM_R, pl.ds(hf, hf)],
                    rs_full_send.at[2], rs_full_recv.at[2], left)

        wait_recv(rs_half_ref.at[FROM_R], rs_half_recv.at[FROM_R])
        lo = left * ch
        acc_ref[pl.ds(lo, hf), :] = (
            acc_ref[pl.ds(lo, hf), :].astype(jnp.float32)
            + rs_half_ref[FROM_R].astype(jnp.float32)
        ).astype(jnp.bfloat16)
        remote_copy(acc_ref.at[pl.ds(lo, hf), :],
                    rs_full_ref.at[FROM_R, pl.ds(0, hf)],
                    rs_full_send.at[3], rs_full_recv.at[3], left)

        z = partial_chunk(my)

        for s in range(4):
            sz = rs_full_ref.at[FROM_L, pl.ds(0, hf)]
            wait_recv(sz, rs_full_recv.at[s])
        z = (z + rs_full_ref[FROM_L].astype(jnp.float32)
             + rs_full_ref[FROM_R].astype(jnp.float32))
        z = z * (1.0 / (1.0 + jnp.exp(-z)))
        out_ref[pl.ds(my * ch, ch), :] = z.astype(jnp.bfloat16)

        remote_copy(out_ref.at[pl.ds(my * ch, ch), :],
                    out_ref.at[pl.ds(my * ch, ch), :],
                    ag_full_send.at[0], ag_full_recv.at[FROM_R], left)
        remote_copy(out_ref.at[pl.ds(my * ch, ch), :],
                    out_ref.at[pl.ds(my * ch, ch), :],
                    ag_full_send.at[1], ag_full_recv.at[FROM_L], right)

        wait_recv(out_ref.at[pl.ds(left * ch, ch), :], ag_full_recv.at[FROM_L])
        remote_copy(out_ref.at[pl.ds(left * ch, hf), :],
                    out_ref.at[pl.ds(left * ch, hf), :],
                    ag_half_send.at[0], ag_half_recv.at[FROM_L], right)

        wait_recv(out_ref.at[pl.ds(right * ch, ch), :], ag_full_recv.at[FROM_R])
        remote_copy(out_ref.at[pl.ds(right * ch + hf, hf), :],
                    out_ref.at[pl.ds(right * ch + hf, hf), :],
                    ag_half_send.at[1], ag_half_recv.at[FROM_R], left)

        wait_recv(out_ref.at[pl.ds(diag * ch, hf), :], ag_half_recv.at[FROM_L])
        wait_recv(out_ref.at[pl.ds(diag * ch + hf, hf), :],
                  ag_half_recv.at[FROM_R])

        for rdma in sends:
            rdma.wait_send()

    return pl.pallas_call(
        body,
        out_shape=jax.ShapeDtypeStruct((m, n), jnp.bfloat16),
        in_specs=[
            pl.BlockSpec(memory_space=pltpu.VMEM),
            pl.BlockSpec(memory_space=pltpu.VMEM),
        ],
        out_specs=pl.BlockSpec(memory_space=pltpu.VMEM),
        scratch_shapes=[
            pltpu.VMEM((m, n), jnp.bfloat16),
            pltpu.VMEM((2, hf, n), jnp.bfloat16),
            pltpu.VMEM((2, ch, n), jnp.bfloat16),
            pltpu.SemaphoreType.DMA((2,)),
            pltpu.SemaphoreType.DMA((2,)),
            pltpu.SemaphoreType.DMA((4,)),
            pltpu.SemaphoreType.DMA((4,)),
            pltpu.SemaphoreType.DMA((2,)),
            pltpu.SemaphoreType.DMA((2,)),
            pltpu.SemaphoreType.DMA((2,)),
            pltpu.SemaphoreType.DMA((2,)),
        ],
        compiler_params=pltpu.CompilerParams(collective_id=0),
    )(A, B)


# device time: 30399 ns/iter; 1.0658x vs baseline; 1.0658x over previous
import jax
import jax.numpy as jnp
from jax import lax
from jax.experimental import pallas as pl
from jax.experimental.pallas import tpu as pltpu

N_DEV = 4


def kernel(A, B):
    m, _ = A.shape
    _, n = B.shape
    ch = m // N_DEV
    hf = ch // 2

    FROM_L, FROM_R = 0, 1

    def body(a_ref, b_ref, out_ref, acc_ref,
             rs_half_ref, rs_full_ref,
             rs_half_send, rs_half_recv, rs_full_send, rs_full_recv,
             ag_full_send, ag_full_recv, ag_half_send, ag_half_recv):
        my = lax.axis_index("i")
        left = (my + N_DEV - 1) % N_DEV
        right = (my + 1) % N_DEV
        diag = (my + 2) % N_DEV

        barrier_sem = pltpu.get_barrier_semaphore()
        for nbr in (left, right):
            pl.semaphore_signal(
                barrier_sem, inc=1,
                device_id=(nbr,), device_id_type=pl.DeviceIdType.MESH,
            )
        pl.semaphore_wait(barrier_sem, 2)

        b16 = b_ref[...].astype(jnp.bfloat16)

        def partial_chunk(p):
            return jnp.dot(
                a_ref[pl.ds(p * ch, ch), :].astype(jnp.bfloat16),
                b16,
                preferred_element_type=jnp.float32,
            )

        sends = []

        def remote_copy(src, dst, ssem, rsem, dev):
            rdma = pltpu.make_async_remote_copy(
                src_ref=src, dst_ref=dst, send_sem=ssem, recv_sem=rsem,
                device_id=(dev,), device_id_type=pl.DeviceIdType.MESH,
            )
            rdma.start()
            sends.append(rdma)

        def wait_recv(buf, sem):
            pltpu.make_async_remote_copy(
                src_ref=buf, dst_ref=buf, send_sem=sem, recv_sem=sem,
                device_id=(my,), device_id_type=pl.DeviceIdType.MESH,
            ).wait_recv()

        acc_ref[pl.ds(diag * ch, ch), :] = partial_chunk(diag).astype(
            jnp.bfloat16)
        remote_copy(acc_ref.at[pl.ds(diag * ch, hf), :],
                    rs_half_ref.at[FROM_R],
                    rs_half_send.at[0], rs_half_recv.at[FROM_R], left)
        remote_copy(acc_ref.at[pl.ds(diag * ch + hf, hf), :],
                    rs_half_ref.at[FROM_L],
                    rs_half_send.at[1], rs_half_recv.at[FROM_L], right)

        acc_ref[pl.ds(right * ch, ch), :] = partial_chunk(right).astype(
            jnp.bfloat16)
        remote_copy(acc_ref.at[pl.ds(right * ch, hf), :],
                    rs_full_ref.at[FROM_L, pl.ds(0, hf)],
                    rs_full_send.at[0], rs_full_recv.at[0], right)

        wait_recv(rs_half_ref.at[FROM_L], rs_half_recv.at[FROM_L])
        hi = right * ch + hf
        acc_ref[pl.ds(hi, hf), :] = (
            acc_ref[pl.ds(hi, hf), :].astype(jnp.float32)
            + rs_half_ref[FROM_L].astype(jnp.float32)
        ).astype(jnp.bfloat16)
        remote_copy(acc_ref.at[pl.ds(hi, hf), :],
                    rs_full_ref.at[FROM_L, pl.ds(hf, hf)],
                    rs_full_send.at[1], rs_full_recv.at[1], right)

        acc_ref[pl.ds(left * ch, ch), :] = partial_chunk(left).astype(
            jnp.bfloat16)
        remote_copy(acc_ref.at[pl.ds(left * ch + hf, hf), :],
                    rs_full_ref.at[FROM_R, pl.ds(hf, hf)],
                    rs_full_send.at[2], rs_full_recv.at[2], left)

        wait_recv(rs_half_ref.at[FROM_R], rs_half_recv.at[FROM_R])
        lo = left * ch
        acc_ref[pl.ds(lo, hf), :] = (
            acc_ref[pl.ds(lo, hf), :].astype(jnp.float32)
            + rs_half_ref[FROM_R].astype(jnp.float32)
        ).astype(jnp.bfloat16)
        remote_copy(acc_ref.at[pl.ds(lo, hf), :],
                    rs_full_ref.at[FROM_R, pl.ds(0, hf)],
                    rs_full_send.at[3], rs_full_recv.at[3], left)

        z = partial_chunk(my)

        sz = rs_full_ref.at[FROM_L, pl.ds(0, hf)]

        wait_recv(sz, rs_full_recv.at[1])
        wait_recv(sz, rs_full_recv.at[2])
        zh = (z[hf:, :]
              + rs_full_ref[FROM_L, pl.ds(hf, hf)].astype(jnp.float32)
              + rs_full_ref[FROM_R, pl.ds(hf, hf)].astype(jnp.float32))
        zh = zh * (1.0 / (1.0 + jnp.exp(-zh)))
        out_ref[pl.ds(my * ch + hf, hf), :] = zh.astype(jnp.bfloat16)
        remote_copy(out_ref.at[pl.ds(my * ch + hf, hf), :],
                    out_ref.at[pl.ds(my * ch + hf, hf), :],
                    ag_full_send.at[1], ag_full_recv.at[2], left)
        remote_copy(out_ref.at[pl.ds(my * ch + hf, hf), :],
                    out_ref.at[pl.ds(my * ch + hf, hf), :],
                    ag_full_send.at[2], ag_full_recv.at[1], right)

        wait_recv(sz, rs_full_recv.at[0])
        wait_recv(sz, rs_full_recv.at[3])
        zl = (z[:hf, :]
              + rs_full_ref[FROM_L, pl.ds(0, hf)].astype(jnp.float32)
              + rs_full_ref[FROM_R, pl.ds(0, hf)].astype(jnp.float32))
        zl = zl * (1.0 / (1.0 + jnp.exp(-zl)))
        out_ref[pl.ds(my * ch, hf), :] = zl.astype(jnp.bfloat16)
        remote_copy(out_ref.at[pl.ds(my * ch, hf), :],
                    out_ref.at[pl.ds(my * ch, hf), :],
                    ag_full_send.at[0], ag_full_recv.at[0], right)
        remote_copy(out_ref.at[pl.ds(my * ch, hf), :],
                    out_ref.at[pl.ds(my * ch, hf), :],
                    ag_full_send.at[3], ag_full_recv.at[3], left)

        wait_recv(out_ref.at[pl.ds(right * ch + hf, hf), :],
                  ag_full_recv.at[2])
        remote_copy(out_ref.at[pl.ds(right * ch + hf, hf), :],
                    out_ref.at[pl.ds(right * ch + hf, hf), :],
                    ag_half_send.at[1], ag_half_recv.at[FROM_R], left)

        wait_recv(out_ref.at[pl.ds(left * ch, hf), :], ag_full_recv.at[0])
        remote_copy(out_ref.at[pl.ds(left * ch, hf), :],
                    out_ref.at[pl.ds(left * ch, hf), :],
                    ag_half_send.at[0], ag_half_recv.at[FROM_L], right)

        wait_recv(out_ref.at[pl.ds(left * ch + hf, hf), :], ag_full_recv.at[1])
        wait_recv(out_ref.at[pl.ds(right * ch, hf), :], ag_full_recv.at[3])
        wait_recv(out_ref.at[pl.ds(diag * ch, hf), :], ag_half_recv.at[FROM_L])
        wait_recv(out_ref.at[pl.ds(diag * ch + hf, hf), :],
                  ag_half_recv.at[FROM_R])

        for rdma in sends:
            rdma.wait_send()

    return pl.pallas_call(
        body,
        out_shape=jax.ShapeDtypeStruct((m, n), jnp.bfloat16),
        in_specs=[
            pl.BlockSpec(memory_space=pltpu.VMEM),
            pl.BlockSpec(memory_space=pltpu.VMEM),
        ],
        out_specs=pl.BlockSpec(memory_space=pltpu.VMEM),
        scratch_shapes=[
            pltpu.VMEM((m, n), jnp.bfloat16),
            pltpu.VMEM((2, hf, n), jnp.bfloat16),
            pltpu.VMEM((2, ch, n), jnp.bfloat16),
            pltpu.SemaphoreType.DMA((2,)),
            pltpu.SemaphoreType.DMA((2,)),
            pltpu.SemaphoreType.DMA((4,)),
            pltpu.SemaphoreType.DMA((4,)),
            pltpu.SemaphoreType.DMA((4,)),
            pltpu.SemaphoreType.DMA((4,)),
            pltpu.SemaphoreType.DMA((2,)),
            pltpu.SemaphoreType.DMA((2,)),
        ],
        compiler_params=pltpu.CompilerParams(collective_id=0),
    )(A, B)
